# Initial kernel scaffold; baseline (speedup 1.0000x reference)
import jax
import jax.numpy as jnp
from jax.experimental import pallas as pl


def kernel(encoder_outputs, graph_coords, batch):
    raise NotImplementedError("write your pallas kernel here")


# speedup vs baseline: 1.0892x; 1.0892x over previous
import functools

import jax
import jax.numpy as jnp
from jax import lax
from jax.experimental import pallas as pl
from jax.experimental.pallas import tpu as pltpu
from jax.experimental.pallas import tpu_sc as plsc

_D, _H, _W = 32, 32, 32
_C = 128
_NC, _NS = 2, 16
_NW = _NC * _NS
_CH = 64
_L = 16


def _axis_prep(coord, dimlen):
    g = 2.0 * coord / (dimlen - 1.0) - 1.0
    v = jnp.clip((g + 1.0) * 0.5 * (dimlen - 1.0), 0.0, dimlen - 1.0)
    i0 = v.astype(jnp.int32)
    w = v - i0.astype(jnp.float32)
    i1 = jnp.minimum(i0 + 1, dimlen - 1)
    return i0, i1, w


def _make_sc_kernel(n_chunks):
    n_pad = _NW * _CH * n_chunks
    mesh = plsc.VectorSubcoreMesh(core_axis_name="c", subcore_axis_name="s")

    scratch = (
        [pltpu.VMEM((_CH,), jnp.float32) for _ in range(3)]
        + [pltpu.VMEM((_CH,), jnp.int32)]
        + [pltpu.VMEM((_CH,), jnp.int32) for _ in range(8)]
        + [pltpu.VMEM((_CH + _L,), jnp.float32) for _ in range(8)]
        + [pltpu.VMEM((_CH, _C), jnp.float32) for _ in range(8)]
        + [pltpu.VMEM((_CH, _C), jnp.float32)]
        + [pltpu.SemaphoreType.DMA]
    )

    @functools.partial(
        pl.kernel,
        mesh=mesh,
        out_type=jax.ShapeDtypeStruct((n_pad, _C), jnp.float32),
        scratch_types=scratch,
    )
    def sc_kernel(xs_h, ys_h, zs_h, bs_h, table_h, out_h, *refs):
        xv, yv, zv = refs[0:3]
        bv = refs[3]
        idx = refs[4:12]
        wgt = refs[12:20]
        rows = refs[20:28]
        ov = refs[28]
        sem = refs[29]

        wid = lax.axis_index("s") * _NC + lax.axis_index("c")
        wbase = wid * (_CH * n_chunks)

        def chunk_body(gi, carry):
            base = pl.multiple_of(wbase + gi * _CH, _CH)
            pltpu.sync_copy(xs_h.at[pl.ds(base, _CH)], xv)
            pltpu.sync_copy(ys_h.at[pl.ds(base, _CH)], yv)
            pltpu.sync_copy(zs_h.at[pl.ds(base, _CH)], zv)
            pltpu.sync_copy(bs_h.at[pl.ds(base, _CH)], bv)

            for i in range(_CH // _L):
                sl = pl.ds(i * _L, _L)
                x0, x1, wx = _axis_prep(xv[sl], _W)
                y0, y1, wy = _axis_prep(yv[sl], _H)
                z0, z1, wz = _axis_prep(zv[sl], _D)
                bb = bv[sl]
                bz0 = (bb * _D + z0) * _H
                bz1 = (bb * _D + z1) * _H
                r00 = (bz0 + y0) * _W
                r01 = (bz0 + y1) * _W
                r10 = (bz1 + y0) * _W
                r11 = (bz1 + y1) * _W
                idx[0][sl] = r00 + x0
                idx[1][sl] = r00 + x1
                idx[2][sl] = r01 + x0
                idx[3][sl] = r01 + x1
                idx[4][sl] = r10 + x0
                idx[5][sl] = r10 + x1
                idx[6][sl] = r11 + x0
                idx[7][sl] = r11 + x1
                ux = 1.0 - wx
                uy = 1.0 - wy
                uz = 1.0 - wz
                wgt[0][sl] = uz * uy * ux
                wgt[1][sl] = uz * uy * wx
                wgt[2][sl] = uz * wy * ux
                wgt[3][sl] = uz * wy * wx
                wgt[4][sl] = wz * uy * ux
                wgt[5][sl] = wz * uy * wx
                wgt[6][sl] = wz * wy * ux
                wgt[7][sl] = wz * wy * wx

            handles = [
                pltpu.async_copy(table_h.at[idx[k]], rows[k], sem)
                for k in range(8)
            ]
            for h in handles:
                h.wait()

            def node_body(nn, c2):
                w8 = [wgt[k][pl.ds(nn, _L)][0] for k in range(8)]
                for j in range(_C // _L):
                    s = pl.ds(j * _L, _L)
                    acc = rows[0][nn, s] * w8[0]
                    for k in range(1, 8):
                        acc = acc + rows[k][nn, s] * w8[k]
                    ov[nn, s] = acc
                return c2

            lax.fori_loop(0, _CH, node_body, 0, unroll=False)
            pltpu.sync_copy(ov, out_h.at[pl.ds(base, _CH)])
            return carry

        lax.fori_loop(0, n_chunks, chunk_body, 0, unroll=False)

    return sc_kernel


def kernel(encoder_outputs, graph_coords, batch):
    n = graph_coords.shape[0]
    b, c = encoder_outputs.shape[0], encoder_outputs.shape[1]
    per_super = _NW * _CH
    n_chunks = -(-n // per_super)
    n_pad = per_super * n_chunks
    pad = n_pad - n

    table = jnp.transpose(encoder_outputs, (0, 2, 3, 4, 1)).reshape(
        b * _D * _H * _W, c)
    xs = jnp.pad(graph_coords[:, 0], (0, pad))
    ys = jnp.pad(graph_coords[:, 1], (0, pad))
    zs = jnp.pad(graph_coords[:, 2], (0, pad))
    bs = jnp.pad(batch, (0, pad))

    out = _make_sc_kernel(n_chunks)(xs, ys, zs, bs, table)
    return out[:n]


# speedup vs baseline: 1.3545x; 1.2436x over previous
import functools

import jax
import jax.numpy as jnp
from jax import lax
from jax.experimental import pallas as pl
from jax.experimental.pallas import tpu as pltpu
from jax.experimental.pallas import tpu_sc as plsc

_D, _H, _W = 32, 32, 32
_C = 128
_NC, _NS = 2, 16
_NW = _NC * _NS
_CH = 48
_L = 16


def _axis_prep(coord, dimlen):
    g = 2.0 * coord / (dimlen - 1.0) - 1.0
    v = jnp.clip((g + 1.0) * 0.5 * (dimlen - 1.0), 0.0, dimlen - 1.0)
    i0 = v.astype(jnp.int32)
    w = v - i0.astype(jnp.float32)
    i1 = jnp.minimum(i0 + 1, dimlen - 1)
    return i0, i1, w


def _make_sc_kernel(n_chunks):
    n_pad = _NW * _CH * n_chunks
    mesh = plsc.VectorSubcoreMesh(core_axis_name="c", subcore_axis_name="s")

    assert n_chunks % 2 == 0
    scratch = (
        [pltpu.VMEM((_CH,), jnp.float32) for _ in range(3)]
        + [pltpu.VMEM((_CH,), jnp.int32)]
        + [pltpu.VMEM((_CH,), jnp.int32) for _ in range(16)]
        + [pltpu.VMEM((_CH + _L,), jnp.float32) for _ in range(16)]
        + [pltpu.VMEM((_CH, _C), jnp.float32) for _ in range(16)]
        + [pltpu.VMEM((_CH, _C), jnp.float32)]
        + [pltpu.SemaphoreType.DMA, pltpu.SemaphoreType.DMA]
    )

    @functools.partial(
        pl.kernel,
        mesh=mesh,
        out_type=jax.ShapeDtypeStruct((n_pad, _C), jnp.float32),
        scratch_types=scratch,
    )
    def sc_kernel(xs_h, ys_h, zs_h, bs_h, table_h, out_h, *refs):
        xv, yv, zv = refs[0:3]
        bv = refs[3]
        idx = (refs[4:12], refs[12:20])
        wgt = (refs[20:28], refs[28:36])
        rows = (refs[36:44], refs[44:52])
        ov = refs[52]
        sem = refs[53:55]

        wid = lax.axis_index("s") * _NC + lax.axis_index("c")
        wbase = wid * (_CH * n_chunks)

        def load_and_fire(gi, s):
            base = pl.multiple_of(wbase + gi * _CH, 8)
            pltpu.sync_copy(xs_h.at[pl.ds(base, _CH)], xv)
            pltpu.sync_copy(ys_h.at[pl.ds(base, _CH)], yv)
            pltpu.sync_copy(zs_h.at[pl.ds(base, _CH)], zv)
            pltpu.sync_copy(bs_h.at[pl.ds(base, _CH)], bv)

            for i in range(_CH // _L):
                sl = pl.ds(i * _L, _L)
                x0, x1, wx = _axis_prep(xv[sl], _W)
                y0, y1, wy = _axis_prep(yv[sl], _H)
                z0, z1, wz = _axis_prep(zv[sl], _D)
                bb = bv[sl]
                bz0 = (bb * _D + z0) * _H
                bz1 = (bb * _D + z1) * _H
                r00 = (bz0 + y0) * _W
                r01 = (bz0 + y1) * _W
                r10 = (bz1 + y0) * _W
                r11 = (bz1 + y1) * _W
                idx[s][0][sl] = r00 + x0
                idx[s][1][sl] = r00 + x1
                idx[s][2][sl] = r01 + x0
                idx[s][3][sl] = r01 + x1
                idx[s][4][sl] = r10 + x0
                idx[s][5][sl] = r10 + x1
                idx[s][6][sl] = r11 + x0
                idx[s][7][sl] = r11 + x1
                ux = 1.0 - wx
                uy = 1.0 - wy
                uz = 1.0 - wz
                wgt[s][0][sl] = uz * uy * ux
                wgt[s][1][sl] = uz * uy * wx
                wgt[s][2][sl] = uz * wy * ux
                wgt[s][3][sl] = uz * wy * wx
                wgt[s][4][sl] = wz * uy * ux
                wgt[s][5][sl] = wz * uy * wx
                wgt[s][6][sl] = wz * wy * ux
                wgt[s][7][sl] = wz * wy * wx

            for k in range(8):
                pltpu.async_copy(table_h.at[idx[s][k]], rows[s][k], sem[s])

        def drain(s):
            for k in range(8):
                pltpu.make_async_copy(
                    table_h.at[idx[s][k]], rows[s][k], sem[s]).wait()

        def accumulate(gi, s):
            base = pl.multiple_of(wbase + gi * _CH, 8)

            def node_body(nn, c2):
                w8 = [wgt[s][k][pl.ds(nn, _L)][0] for k in range(8)]
                for j in range(_C // _L):
                    sj = pl.ds(j * _L, _L)
                    acc = rows[s][0][nn, sj] * w8[0]
                    for k in range(1, 8):
                        acc = acc + rows[s][k][nn, sj] * w8[k]
                    ov[nn, sj] = acc
                return c2

            lax.fori_loop(0, _CH, node_body, 0, unroll=False)
            pltpu.sync_copy(ov, out_h.at[pl.ds(base, _CH)])

        load_and_fire(0, 0)

        def outer(gp, carry):
            for b in range(2):
                g = 2 * gp + b
                nxt = g + 1

                @pl.when(nxt < n_chunks)
                def _():
                    load_and_fire(nxt, (b + 1) % 2)

                drain(b)
                accumulate(g, b)
            return carry

        lax.fori_loop(0, n_chunks // 2, outer, 0, unroll=False)

    return sc_kernel


def kernel(encoder_outputs, graph_coords, batch):
    n = graph_coords.shape[0]
    b, c = encoder_outputs.shape[0], encoder_outputs.shape[1]
    per_super = _NW * _CH
    n_chunks = -(-n // per_super)
    n_chunks += n_chunks % 2
    n_pad = per_super * n_chunks
    pad = n_pad - n

    table = jnp.transpose(encoder_outputs, (0, 2, 3, 4, 1)).reshape(
        b * _D * _H * _W, c)
    xs = jnp.pad(graph_coords[:, 0], (0, pad))
    ys = jnp.pad(graph_coords[:, 1], (0, pad))
    zs = jnp.pad(graph_coords[:, 2], (0, pad))
    bs = jnp.pad(batch, (0, pad))

    out = _make_sc_kernel(n_chunks)(xs, ys, zs, bs, table)
    return out[:n]
